# baseline (speedup 1.0000x reference)
import jax
import jax.numpy as jnp
from jax.experimental import pallas as pl
from jax.experimental.pallas import tpu as pltpu

_BLK = 2048
_EPS = 1e-5


def _ln_body(x_ref, pos_ref, g_ref, b_ref, o_ref):
    h = x_ref[...] + pos_ref[...]
    mean = jnp.mean(h, axis=-1, keepdims=True)
    d = h - mean
    var = jnp.mean(d * d, axis=-1, keepdims=True)
    o_ref[...] = d * jax.lax.rsqrt(var + _EPS) * g_ref[...] + b_ref[...]


def kernel(x, pos_table, gamma, beta):
    B, S, D = x.shape
    xf = x.reshape(B * S, D)
    gamma2 = gamma.reshape(1, D)
    beta2 = beta.reshape(1, D)
    nseg = S // _BLK
    grid = (B * S // _BLK,)

    def x_map(j):
        return (j // B + (j % B) * nseg, 0)

    def pos_map(j):
        return (j // B, 0)

    out = pl.pallas_call(
        _ln_body,
        grid=grid,
        in_specs=[
            pl.BlockSpec((_BLK, D), x_map),
            pl.BlockSpec((_BLK, D), pos_map),
            pl.BlockSpec((1, D), lambda j: (0, 0)),
            pl.BlockSpec((1, D), lambda j: (0, 0)),
        ],
        out_specs=pl.BlockSpec((_BLK, D), x_map),
        out_shape=jax.ShapeDtypeStruct((B * S, D), x.dtype),
        compiler_params=pltpu.CompilerParams(
            dimension_semantics=("arbitrary",),
        ),
    )(xf, pos_table, gamma2, beta2)
    return out.reshape(B, S, D)
